# Initial kernel scaffold; baseline (speedup 1.0000x reference)
import jax
import jax.numpy as jnp
from jax.experimental import pallas as pl


def kernel(x, W):
    raise NotImplementedError("write your pallas kernel here")


# speedup vs baseline: 9.0198x; 9.0198x over previous
import functools

import jax
import jax.numpy as jnp
from jax import lax
from jax.experimental import pallas as pl
from jax.experimental.pallas import tpu as pltpu

HIDDEN = 768
NUM_EXPERTS = 64

_TOKEN_TILE = 1024


def _matmul_kernel(x_ref, w_ref, lt_ref, rl_ref):
    lt = lax.dot_general(
        w_ref[...], x_ref[...],
        dimension_numbers=(((1,), (1,)), ((), ())),
        preferred_element_type=jnp.float32,
    )
    lt_ref[...] = lt
    rl_ref[...] = lt.T


def _select_kernel(lt_ref, score_ref, idx_ref, counts_ref, *, n_tokens, k):
    E = NUM_EXPERTS
    N = n_tokens
    SIGN = jnp.int32(-2147483648)
    lt = lt_ref[...]
    bits = lax.bitcast_convert_type(lt, jnp.int32)
    ukey = jnp.where(bits >= 0, bits | SIGN, ~bits)
    okey = ukey ^ SIGN

    def bit_step(i, carry):
        prefix, kk = carry
        b = jnp.int32(31) - i
        target = lax.shift_right_logical(prefix, b) | jnp.int32(1)
        match = lax.shift_right_logical(ukey, b) == target
        c1 = jnp.sum(match.astype(jnp.int32), axis=1, keepdims=True)
        take_hi = kk <= c1
        prefix = jnp.where(take_hi, prefix | lax.shift_left(jnp.int32(1), b),
                           prefix)
        kk = jnp.where(take_hi, kk, kk - c1)
        return prefix, kk

    prefix0 = jnp.zeros((E, 1), jnp.int32)
    kk0 = jnp.full((E, 1), k, jnp.int32)
    K, _ = lax.fori_loop(0, 32, bit_step, (prefix0, kk0))

    oK = K ^ SIGN
    c_gt = jnp.sum((okey > oK).astype(jnp.int32), axis=1, keepdims=True)
    need = k - c_gt

    tok = lax.broadcasted_iota(jnp.int32, (E, N), 1)
    is_k = ukey == K

    def idx_step(i, carry):
        tpos, rem = carry
        span = jnp.int32(1) << (jnp.int32(14) - i)
        inblk = is_k & (tok >= tpos) & (tok < tpos + span)
        c = jnp.sum(inblk.astype(jnp.int32), axis=1, keepdims=True)
        adv = rem > c
        tpos = jnp.where(adv, tpos + span, tpos)
        rem = jnp.where(adv, rem - c, rem)
        return tpos, rem

    tpos0 = jnp.zeros((E, 1), jnp.int32)
    idx_thr, _ = lax.fori_loop(0, 15, idx_step, (tpos0, need))

    mask = (okey > oK) | (is_k & (tok <= idx_thr))

    selected = jnp.any(mask, axis=0, keepdims=True)
    eff_mask = mask | ~selected
    cand = jnp.where(eff_mask, okey, SIGN)
    best = jnp.max(cand, axis=0, keepdims=True)
    eidx = lax.broadcasted_iota(jnp.int32, (E, N), 0)
    expert = jnp.min(
        jnp.where(eff_mask & (cand == best), eidx, jnp.int32(E)),
        axis=0, keepdims=True,
    )

    bb = jnp.where(best >= 0, best, ~(best ^ SIGN))
    score = lax.bitcast_convert_type(bb, jnp.float32)

    score_ref[...] = score
    idx_ref[...] = expert
    counts_ref[...] = jnp.sum(
        (expert == lax.broadcasted_iota(jnp.int32, (E, N), 0)).astype(jnp.float32),
        axis=1, keepdims=True,
    )


def _run(x, W, interpret=False):
    B, S, H = x.shape
    E = W.shape[0]
    N = B * S
    k = max(1, min(N // E, N))
    x_flat = x.reshape(N, H)
    n_tiles = N // _TOKEN_TILE

    lt, router_logits = pl.pallas_call(
        _matmul_kernel,
        grid=(n_tiles,),
        in_specs=[
            pl.BlockSpec((_TOKEN_TILE, H), lambda i: (i, 0)),
            pl.BlockSpec((E, H), lambda i: (0, 0)),
        ],
        out_specs=[
            pl.BlockSpec((E, _TOKEN_TILE), lambda i: (0, i)),
            pl.BlockSpec((_TOKEN_TILE, E), lambda i: (i, 0)),
        ],
        out_shape=[
            jax.ShapeDtypeStruct((E, N), jnp.float32),
            jax.ShapeDtypeStruct((N, E), jnp.float32),
        ],
        interpret=interpret,
    )(x_flat, W)

    score, expert, counts = pl.pallas_call(
        functools.partial(_select_kernel, n_tokens=N, k=k),
        out_shape=[
            jax.ShapeDtypeStruct((1, N), jnp.float32),
            jax.ShapeDtypeStruct((1, N), jnp.int32),
            jax.ShapeDtypeStruct((E, 1), jnp.float32),
        ],
        interpret=interpret,
    )(lt)

    routing_weights = score.reshape(B, S)
    expert_indices = expert.reshape(B, S)
    expert_counts = counts.reshape(E)
    aux_loss = jnp.float32(0.0)
    return routing_weights, expert_indices, router_logits, aux_loss, expert_counts


def kernel(x, W):
    return _run(x, W, interpret=False)
